# baseline (speedup 1.0000x reference)
import functools

import jax
import jax.numpy as jnp
from jax import lax
from jax.experimental import pallas as pl
from jax.experimental.pallas import tpu as pltpu
from jax.experimental.pallas import tpu_sc as plsc

B = 64
C_IN = 384
C_OUT = 192
HW = 48 * 48
NC, NS, L = 2, 16, 16
NW = NC * NS
ROWS_TOTAL = B * C_OUT
RPT = ROWS_TOTAL // NW
CH = 24
NG = RPT // CH
BPT = RPT // C_OUT


def _sc_gather(x2d, filt):
    mesh = plsc.VectorSubcoreMesh(core_axis_name="c", subcore_axis_name="s")

    @functools.partial(
        pl.kernel,
        mesh=mesh,
        out_type=jax.ShapeDtypeStruct((ROWS_TOTAL, HW), jnp.float32),
        scratch_types=[
            pltpu.VMEM((C_OUT,), jnp.int32),
            pltpu.VMEM((RPT,), jnp.int32),
            pltpu.VMEM((2, CH, HW), jnp.float32),
            pltpu.SemaphoreType.DMA,
            pltpu.SemaphoreType.DMA,
            pltpu.SemaphoreType.DMA,
            pltpu.SemaphoreType.DMA,
        ],
    )
    def k(x_hbm, filt_hbm, out_hbm, filt_v, idx_v, buf, g0, g1, p0, p1):
        wid = lax.axis_index("s") * NC + lax.axis_index("c")
        pltpu.sync_copy(filt_hbm, filt_v)
        for kk in range(RPT // L):
            j = kk // (C_OUT // L)
            c0 = (kk % (C_OUT // L)) * L
            row_base = (BPT * wid + j) * C_IN
            idx_v[pl.ds(kk * L, L)] = filt_v[pl.ds(c0, L)] + row_base

        gsem = (g0, g1)
        psem = (p0, p1)
        base_out = wid * RPT
        gathers = [None, None]
        puts = [None, None]
        cp = pltpu.make_async_copy(
            x_hbm.at[idx_v.at[pl.ds(0, CH)]], buf.at[0], gsem[0])
        cp.start()
        gathers[0] = cp
        for g in range(NG):
            slot = g % 2
            nxt = 1 - slot
            if g + 1 < NG:
                if puts[nxt] is not None:
                    puts[nxt].wait()
                cp = pltpu.make_async_copy(
                    x_hbm.at[idx_v.at[pl.ds((g + 1) * CH, CH)]],
                    buf.at[nxt], gsem[nxt])
                cp.start()
                gathers[nxt] = cp
            gathers[slot].wait()
            cp = pltpu.make_async_copy(
                buf.at[slot], out_hbm.at[pl.ds(base_out + g * CH, CH)],
                psem[slot])
            cp.start()
            puts[slot] = cp
        puts[(NG - 2) % 2].wait()
        puts[(NG - 1) % 2].wait()

    return k(x2d, filt)


def kernel(x, filters_to_keep):
    x2d = x.reshape(B * C_IN, HW)
    out = _sc_gather(x2d, filters_to_keep.astype(jnp.int32))
    return out.reshape(B, C_OUT, 48, 48)


# speedup vs baseline: 4.7437x; 4.7437x over previous
import functools

import jax
import jax.numpy as jnp
from jax import lax
from jax.experimental import pallas as pl
from jax.experimental.pallas import tpu as pltpu
from jax.experimental.pallas import tpu_sc as plsc

B = 64
C_IN = 384
C_OUT = 192
H = W = 48
L = 16
NC, NS = 2, 16
NW = NC * NS
N = B * H * W
RPT = N // NW
K = 96
NCHUNK = RPT // K
NJ = C_OUT // L


def _sc_gather(x2, filt):
    mesh = plsc.VectorSubcoreMesh(core_axis_name="c", subcore_axis_name="s")

    @functools.partial(
        pl.kernel,
        mesh=mesh,
        out_type=jax.ShapeDtypeStruct((N, C_OUT), jnp.float32),
        scratch_types=[
            pltpu.VMEM((C_OUT,), jnp.int32),
            pltpu.VMEM((K, C_IN), jnp.float32),
            pltpu.VMEM((K, C_IN), jnp.float32),
            pltpu.VMEM((K, C_OUT), jnp.float32),
            pltpu.VMEM((K, C_OUT), jnp.float32),
            pltpu.SemaphoreType.DMA,
            pltpu.SemaphoreType.DMA,
            pltpu.SemaphoreType.DMA,
            pltpu.SemaphoreType.DMA,
        ],
        compiler_params=pltpu.CompilerParams(needs_layout_passes=False),
    )
    def k(x_hbm, filt_hbm, out_hbm, filt_v, ibuf0, ibuf1, obuf0, obuf1,
          gi0, gi1, po0, po1):
        wid = lax.axis_index("s") * NC + lax.axis_index("c")
        base = wid * RPT
        pltpu.sync_copy(filt_hbm, filt_v)
        cols = [filt_v[pl.ds(j * L, L)] for j in range(NJ)]
        ibuf = (ibuf0, ibuf1)
        obuf = (obuf0, obuf1)
        gsem = (gi0, gi1)
        psem = (po0, po1)

        def in_cp(g, s):
            return pltpu.make_async_copy(
                x_hbm.at[pl.ds(base + g * K, K)], ibuf[s], gsem[s])

        def out_cp(g, s):
            return pltpu.make_async_copy(
                obuf[s], out_hbm.at[pl.ds(base + g * K, K)], psem[s])

        in_cp(0, 0).start()
        for g in range(NCHUNK):
            s = g % 2
            if g + 1 < NCHUNK:
                in_cp(g + 1, 1 - s).start()
            in_cp(g, s).wait()
            if g >= 2:
                out_cp(g - 2, s).wait()
            ib = ibuf[s]
            ob = obuf[s]

            @pl.loop(0, K)
            def _row(r):
                rv = jnp.full((L,), r, dtype=jnp.int32)
                for j in range(NJ):
                    ob[r, pl.ds(j * L, L)] = plsc.load_gather(
                        ib, [rv, cols[j]])

            out_cp(g, s).start()
        out_cp(NCHUNK - 2, NCHUNK % 2).wait()
        out_cp(NCHUNK - 1, (NCHUNK - 1) % 2).wait()

    return k(x2, filt)


def kernel(x, filters_to_keep):
    xp = jnp.transpose(x, (0, 2, 3, 1)).reshape(N, C_IN)
    out2 = _sc_gather(xp, filters_to_keep.astype(jnp.int32))
    return jnp.transpose(out2.reshape(B, H, W, C_OUT), (0, 3, 1, 2))


# speedup vs baseline: 5.5489x; 1.1697x over previous
import functools

import jax
import jax.numpy as jnp
from jax import lax
from jax.experimental import pallas as pl
from jax.experimental.pallas import tpu as pltpu
from jax.experimental.pallas import tpu_sc as plsc

B = 64
C_IN = 384
C_OUT = 192
H = W = 48
L = 16
NC, NS = 2, 16
NW = NC * NS
N = B * H * W
RPT = N // NW
K = 96
NCHUNK = RPT // K
NJ = C_OUT // L


def _sc_gather(x2, filt):
    mesh = plsc.VectorSubcoreMesh(core_axis_name="c", subcore_axis_name="s")

    @functools.partial(
        pl.kernel,
        mesh=mesh,
        out_type=jax.ShapeDtypeStruct((N, C_OUT), jnp.float32),
        scratch_types=[
            pltpu.VMEM((C_OUT,), jnp.int32),
            pltpu.VMEM((K, C_IN), jnp.float32),
            pltpu.VMEM((K, C_IN), jnp.float32),
            pltpu.VMEM((K, C_OUT), jnp.float32),
            pltpu.VMEM((K, C_OUT), jnp.float32),
            pltpu.SemaphoreType.DMA,
            pltpu.SemaphoreType.DMA,
            pltpu.SemaphoreType.DMA,
            pltpu.SemaphoreType.DMA,
        ],
        compiler_params=pltpu.CompilerParams(needs_layout_passes=False),
    )
    def k(x_hbm, filt_hbm, out_hbm, filt_v, ibuf0, ibuf1, obuf0, obuf1,
          gi0, gi1, po0, po1):
        wid = lax.axis_index("s") * NC + lax.axis_index("c")
        base = wid * RPT
        pltpu.sync_copy(filt_hbm, filt_v)
        cols = [filt_v[pl.ds(j * L, L)] for j in range(NJ)]
        ibuf = (ibuf0, ibuf1)
        obuf = (obuf0, obuf1)
        gsem = (gi0, gi1)
        psem = (po0, po1)

        def in_cp(g, s):
            return pltpu.make_async_copy(
                x_hbm.at[pl.ds(base + g * K, K)], ibuf[s], gsem[s])

        def out_cp(g, s):
            return pltpu.make_async_copy(
                obuf[s], out_hbm.at[pl.ds(base + g * K, K)], psem[s])

        def compute(ib, ob):
            @pl.loop(0, K, unroll=4)
            def _row(r):
                rv = jnp.full((L,), r, dtype=jnp.int32)
                for j in range(NJ):
                    ob[r, pl.ds(j * L, L)] = plsc.load_gather(
                        ib, [rv, cols[j]])

        in_cp(0, 0).start()

        @pl.loop(0, NCHUNK, step=2)
        def _chunks(g):
            in_cp(g + 1, 1).start()
            in_cp(g, 0).wait()

            @pl.when(g >= 2)
            def _():
                out_cp(g - 2, 0).wait()

            compute(ibuf[0], obuf[0])
            out_cp(g, 0).start()

            @pl.when(g + 2 < NCHUNK)
            def _():
                in_cp(g + 2, 0).start()

            in_cp(g + 1, 1).wait()

            @pl.when(g >= 1)
            def _():
                out_cp(g - 1, 1).wait()

            compute(ibuf[1], obuf[1])
            out_cp(g + 1, 1).start()

        out_cp(NCHUNK - 2, 0).wait()
        out_cp(NCHUNK - 1, 1).wait()

    return k(x2, filt)


def kernel(x, filters_to_keep):
    xp = jnp.transpose(x, (0, 2, 3, 1)).reshape(N, C_IN)
    out2 = _sc_gather(xp, filters_to_keep.astype(jnp.int32))
    return jnp.transpose(out2.reshape(B, H, W, C_OUT), (0, 3, 1, 2))


# speedup vs baseline: 8.2193x; 1.4813x over previous
import functools

import jax
import jax.numpy as jnp
from jax import lax
from jax.experimental import pallas as pl
from jax.experimental.pallas import tpu as pltpu
from jax.experimental.pallas import tpu_sc as plsc

B = 64
C_IN = 384
C_OUT = 192
H = W = 48
L = 16
NC, NS = 2, 16
NW = NC * NS
N = B * H * W
RPT = N // NW
K = 96
NCHUNK = RPT // K
NJ = C_OUT // L


def _sc_gather(x2, filt):
    mesh = plsc.VectorSubcoreMesh(core_axis_name="c", subcore_axis_name="s")

    @functools.partial(
        pl.kernel,
        mesh=mesh,
        out_type=jax.ShapeDtypeStruct((N, C_OUT), jnp.float32),
        scratch_types=[
            pltpu.VMEM((C_OUT,), jnp.int32),
            pltpu.VMEM((K, C_IN), jnp.float32),
            pltpu.VMEM((K, C_IN), jnp.float32),
            pltpu.VMEM((K, C_OUT), jnp.float32),
            pltpu.VMEM((K, C_OUT), jnp.float32),
            pltpu.SemaphoreType.DMA,
            pltpu.SemaphoreType.DMA,
            pltpu.SemaphoreType.DMA,
            pltpu.SemaphoreType.DMA,
        ],
        compiler_params=pltpu.CompilerParams(needs_layout_passes=False),
    )
    def k(x_hbm, filt_hbm, out_hbm, filt_v, ibuf0, ibuf1, obuf0, obuf1,
          gi0, gi1, po0, po1):
        wid = lax.axis_index("s") * NC + lax.axis_index("c")
        base = wid * RPT
        pltpu.sync_copy(filt_hbm, filt_v)
        cols = [filt_v[pl.ds(j * L, L)] for j in range(NJ)]
        ibuf = (ibuf0, ibuf1)
        obuf = (obuf0, obuf1)
        gsem = (gi0, gi1)
        psem = (po0, po1)

        def in_cp(g, s):
            return pltpu.make_async_copy(
                x_hbm.at[pl.ds(base + g * K, K)], ibuf[s], gsem[s])

        def out_cp(g, s):
            return pltpu.make_async_copy(
                obuf[s], out_hbm.at[pl.ds(base + g * K, K)], psem[s])

        def compute(ib, ob):
            @plsc.parallel_loop(0, K, unroll=4)
            def _row(r):
                rv = jnp.full((L,), r, dtype=jnp.int32)
                for j in range(NJ):
                    ob[r, pl.ds(j * L, L)] = plsc.load_gather(
                        ib, [rv, cols[j]])

        in_cp(0, 0).start()

        @pl.loop(0, NCHUNK, step=2)
        def _chunks(g):
            in_cp(g + 1, 1).start()
            in_cp(g, 0).wait()

            @pl.when(g >= 2)
            def _():
                out_cp(g - 2, 0).wait()

            compute(ibuf[0], obuf[0])
            out_cp(g, 0).start()

            @pl.when(g + 2 < NCHUNK)
            def _():
                in_cp(g + 2, 0).start()

            in_cp(g + 1, 1).wait()

            @pl.when(g >= 1)
            def _():
                out_cp(g - 1, 1).wait()

            compute(ibuf[1], obuf[1])
            out_cp(g + 1, 1).start()

        out_cp(NCHUNK - 2, 0).wait()
        out_cp(NCHUNK - 1, 1).wait()

    return k(x2, filt)


def kernel(x, filters_to_keep):
    xp = jnp.transpose(x, (0, 2, 3, 1)).reshape(N, C_IN)
    out2 = _sc_gather(xp, filters_to_keep.astype(jnp.int32))
    return jnp.transpose(out2.reshape(B, H, W, C_OUT), (0, 3, 1, 2))
